# baseline (speedup 1.0000x reference)
import functools
import jax
import jax.numpy as jnp
from jax import lax
from jax.experimental import pallas as pl
from jax.experimental.pallas import tpu as pltpu
from jax.experimental.pallas import tpu_sc as plsc

N = 10000
E = 320000
D_IN = 128
HIDDEN = 128
CLASSES = 64

NCORES = 2
NSUB = 16
NW = NCORES * NSUB
CHUNK = 80
NITER = 126
E_PER = NITER * CHUNK
E_PAD = NW * E_PER
CHUNK_D = 80
NITER_D = E_PER // CHUNK_D
SPT = 626
N_ACC = NSUB * SPT
N_PAD = 10240
DPT = N_PAD // NSUB


_SC_PARAMS = pltpu.CompilerParams(use_tc_tiling_on_sc=False)


def _make_deg_kernel():
    mesh = plsc.VectorSubcoreMesh(core_axis_name="c", subcore_axis_name="s")

    @functools.partial(
        pl.kernel,
        mesh=mesh,
        compiler_params=_SC_PARAMS,
        out_type=jax.ShapeDtypeStruct((NCORES * N_PAD,), jnp.float32),
        scratch_types=[
            pltpu.VMEM_SHARED((N_PAD,), jnp.float32),
            pltpu.VMEM((NITER_D, CHUNK_D), jnp.int32),
            pltpu.VMEM((CHUNK_D,), jnp.float32),
        ],
    )
    def deg_kernel(dstr_hbm, zeros_hbm, out_hbm, acc, dst2d, ones_v):
        c = lax.axis_index("c")
        s = lax.axis_index("s")
        w = c * NSUB + s
        pltpu.sync_copy(zeros_hbm, acc.at[pl.ds(s * DPT, DPT)])
        pltpu.sync_copy(dstr_hbm.at[pl.ds(w * NITER_D, NITER_D)], dst2d)
        for j in range(CHUNK_D // 16):
            ones_v[pl.ds(j * 16, 16)] = jnp.full((16,), 1.0, jnp.float32)
        plsc.subcore_barrier()

        def body(i, carry):
            pltpu.sync_copy(ones_v, acc.at[dst2d.at[i]], add=True)
            return carry

        lax.fori_loop(0, NITER_D, body, 0)
        plsc.subcore_barrier()
        pltpu.sync_copy(
            acc.at[pl.ds(s * DPT, DPT)],
            out_hbm.at[pl.ds(c * N_PAD + s * DPT, DPT)],
        )

    return deg_kernel


def _make_agg_kernel(D):
    mesh = plsc.VectorSubcoreMesh(core_axis_name="c", subcore_axis_name="s")
    NBUF = 3 if D == 128 else 6
    NSUPER = NITER // NBUF

    @functools.partial(
        pl.kernel,
        mesh=mesh,
        compiler_params=_SC_PARAMS,
        out_type=jax.ShapeDtypeStruct((NCORES * N_ACC, D), jnp.float32),
        scratch_types=[
            pltpu.VMEM_SHARED((N_ACC, D), jnp.float32),
            pltpu.VMEM((NITER, CHUNK), jnp.int32),
            pltpu.VMEM((NITER, CHUNK), jnp.int32),
            pltpu.VMEM((NBUF, CHUNK, D), jnp.float32),
            pltpu.SemaphoreType.DMA((NBUF,)),
            pltpu.SemaphoreType.DMA((NBUF,)),
        ],
    )
    def agg_kernel(xs_hbm, srcr_hbm, dstr_hbm, zeros_hbm, out_hbm,
                   acc, src2d, dst2d, rows_v, gsem, ssem):
        c = lax.axis_index("c")
        s = lax.axis_index("s")
        w = c * NSUB + s
        pltpu.sync_copy(zeros_hbm, acc.at[pl.ds(s * SPT, SPT)])
        pltpu.sync_copy(srcr_hbm.at[pl.ds(w * NITER, NITER)], src2d)
        pltpu.sync_copy(dstr_hbm.at[pl.ds(w * NITER, NITER)], dst2d)
        plsc.subcore_barrier()

        def gather(i, b):
            return pltpu.make_async_copy(
                xs_hbm.at[src2d.at[i]], rows_v.at[b], gsem.at[b])

        def scat(i, b):
            return pltpu.make_async_copy(
                rows_v.at[b], acc.at[dst2d.at[i]], ssem.at[b])

        for b in range(NBUF):
            gather(b, b).start()

        def body(g, carry):
            base = g * NBUF
            for b in range(NBUF):
                gather(base + b, b).wait()
                scat(base + b, b).start(add=True)
            for b in range(NBUF):
                scat(base + b, b).wait()
                gather(base + NBUF + b, b).start()
            return carry

        lax.fori_loop(0, NSUPER - 1, body, 0)
        base = NITER - NBUF
        for b in range(NBUF):
            gather(base + b, b).wait()
            scat(base + b, b).start(add=True)
        for b in range(NBUF):
            scat(base + b, b).wait()
        plsc.subcore_barrier()
        pltpu.sync_copy(
            acc.at[pl.ds(s * SPT, SPT)],
            out_hbm.at[pl.ds(c * N_ACC + s * SPT, SPT)],
        )

    return agg_kernel


_deg_call = _make_deg_kernel()
_agg128_call = _make_agg_kernel(HIDDEN)
_agg64_call = _make_agg_kernel(CLASSES)

BM = 2000
GRID = N // BM


def _layer1_body(x_ref, w_ref, d0_ref, d1_ref, xs_ref, dinv_ref):
    deg = d0_ref[...] + d1_ref[...] + 1.0
    dinv = lax.rsqrt(deg)
    h = jnp.dot(x_ref[...], w_ref[...], preferred_element_type=jnp.float32)
    xs_ref[...] = h * dinv
    dinv_ref[...] = dinv


def _tc_layer1(x, W1, d0, d1):
    return pl.pallas_call(
        _layer1_body,
        grid=(GRID,),
        in_specs=[
            pl.BlockSpec((BM, D_IN), lambda i: (i, 0)),
            pl.BlockSpec((D_IN, HIDDEN), lambda i: (0, 0)),
            pl.BlockSpec((BM, 1), lambda i: (i, 0)),
            pl.BlockSpec((BM, 1), lambda i: (i, 0)),
        ],
        out_specs=[
            pl.BlockSpec((BM, HIDDEN), lambda i: (i, 0)),
            pl.BlockSpec((BM, 1), lambda i: (i, 0)),
        ],
        out_shape=[
            jax.ShapeDtypeStruct((N, HIDDEN), jnp.float32),
            jax.ShapeDtypeStruct((N, 1), jnp.float32),
        ],
    )(x, W1, d0, d1)


def _layer2_body(a0_ref, a1_ref, xs_ref, dinv_ref, b_ref, w_ref, out_ref):
    dinv = dinv_ref[...]
    z = dinv * (a0_ref[...] + a1_ref[...] + xs_ref[...]) + b_ref[...]
    o = jnp.maximum(z, 0.0)
    h2 = jnp.dot(o, w_ref[...], preferred_element_type=jnp.float32)
    out_ref[...] = h2 * dinv


def _tc_layer2(a0, a1, xs1, dinv, b1, W2):
    return pl.pallas_call(
        _layer2_body,
        grid=(GRID,),
        in_specs=[
            pl.BlockSpec((BM, HIDDEN), lambda i: (i, 0)),
            pl.BlockSpec((BM, HIDDEN), lambda i: (i, 0)),
            pl.BlockSpec((BM, HIDDEN), lambda i: (i, 0)),
            pl.BlockSpec((BM, 1), lambda i: (i, 0)),
            pl.BlockSpec((1, HIDDEN), lambda i: (0, 0)),
            pl.BlockSpec((HIDDEN, CLASSES), lambda i: (0, 0)),
        ],
        out_specs=pl.BlockSpec((BM, CLASSES), lambda i: (i, 0)),
        out_shape=jax.ShapeDtypeStruct((N, CLASSES), jnp.float32),
    )(a0, a1, xs1, dinv, b1, W2)


def _final_body(a0_ref, a1_ref, xs_ref, dinv_ref, b_ref, out_ref):
    z = dinv_ref[...] * (a0_ref[...] + a1_ref[...] + xs_ref[...]) + b_ref[...]
    out_ref[...] = z


def _tc_final(a0, a1, xs2, dinv, b2):
    return pl.pallas_call(
        _final_body,
        grid=(GRID,),
        in_specs=[
            pl.BlockSpec((BM, CLASSES), lambda i: (i, 0)),
            pl.BlockSpec((BM, CLASSES), lambda i: (i, 0)),
            pl.BlockSpec((BM, CLASSES), lambda i: (i, 0)),
            pl.BlockSpec((BM, 1), lambda i: (i, 0)),
            pl.BlockSpec((1, CLASSES), lambda i: (0, 0)),
        ],
        out_specs=pl.BlockSpec((BM, CLASSES), lambda i: (i, 0)),
        out_shape=jax.ShapeDtypeStruct((N, CLASSES), jnp.float32),
    )(a0, a1, xs2, dinv, b2)


def kernel(data, edge_index, W1, b1, W2, b2):
    ei = edge_index.astype(jnp.int32)
    pad = jnp.full((E_PAD - E,), N, jnp.int32)
    src = jnp.concatenate([ei[0], pad]).reshape(E_PAD // CHUNK, CHUNK)
    dst_flat = jnp.concatenate([ei[1], pad])
    dst = dst_flat.reshape(E_PAD // CHUNK, CHUNK)
    dst_deg = dst_flat.reshape(E_PAD // CHUNK_D, CHUNK_D)
    zrows128 = jnp.zeros((N_ACC - N, HIDDEN), jnp.float32)
    zrows64 = jnp.zeros((N_ACC - N, CLASSES), jnp.float32)

    zeros_deg = jnp.zeros((DPT,), jnp.float32)
    zeros128 = jnp.zeros((SPT, HIDDEN), jnp.float32)
    zeros64 = jnp.zeros((SPT, CLASSES), jnp.float32)

    deg_parts = _deg_call(dst_deg, zeros_deg)
    d0 = deg_parts[0 * N_PAD:0 * N_PAD + N].reshape(N, 1)
    d1 = deg_parts[1 * N_PAD:1 * N_PAD + N].reshape(N, 1)

    xs1, dinv = _tc_layer1(data, W1, d0, d1)

    agg1 = _agg128_call(jnp.concatenate([xs1, zrows128]), src, dst,
                        zeros128)
    xs2 = _tc_layer2(agg1[:N], agg1[N_ACC:N_ACC + N], xs1, dinv,
                     b1.reshape(1, HIDDEN), W2)

    agg2 = _agg64_call(jnp.concatenate([xs2, zrows64]), src, dst,
                       zeros64)
    out = _tc_final(agg2[:N], agg2[N_ACC:N_ACC + N], xs2, dinv,
                    b2.reshape(1, CLASSES))
    return out


# speedup vs baseline: 1.0572x; 1.0572x over previous
import functools
import jax
import jax.numpy as jnp
from jax import lax
from jax.experimental import pallas as pl
from jax.experimental.pallas import tpu as pltpu
from jax.experimental.pallas import tpu_sc as plsc

N = 10000
E = 320000
D_IN = 128
HIDDEN = 128
CLASSES = 64

NCORES = 2
NSUB = 16
NW = NCORES * NSUB
CHUNK = 80
NITER = 126
E_PER = NITER * CHUNK
E_PAD = NW * E_PER
CHUNK_D = 80
NITER_D = E_PER // CHUNK_D
SPT = 626
N_ACC = NSUB * SPT
N_PAD = 10240
DPT = N_PAD // NSUB


_SC_PARAMS = pltpu.CompilerParams(use_tc_tiling_on_sc=False)


def _make_deg_kernel():
    mesh = plsc.VectorSubcoreMesh(core_axis_name="c", subcore_axis_name="s")

    @functools.partial(
        pl.kernel,
        mesh=mesh,
        compiler_params=_SC_PARAMS,
        out_type=jax.ShapeDtypeStruct((NCORES * N_PAD,), jnp.float32),
        scratch_types=[
            pltpu.VMEM_SHARED((N_PAD,), jnp.float32),
            pltpu.VMEM((NITER_D, CHUNK_D), jnp.int32),
            pltpu.VMEM((CHUNK_D,), jnp.float32),
        ],
    )
    def deg_kernel(dstr_hbm, zeros_hbm, out_hbm, acc, dst2d, ones_v):
        c = lax.axis_index("c")
        s = lax.axis_index("s")
        w = c * NSUB + s
        pltpu.sync_copy(zeros_hbm, acc.at[pl.ds(s * DPT, DPT)])
        pltpu.sync_copy(dstr_hbm.at[pl.ds(w * NITER_D, NITER_D)], dst2d)
        for j in range(CHUNK_D // 16):
            ones_v[pl.ds(j * 16, 16)] = jnp.full((16,), 1.0, jnp.float32)
        plsc.subcore_barrier()

        def body(i, carry):
            pltpu.sync_copy(ones_v, acc.at[dst2d.at[i]], add=True)
            return carry

        lax.fori_loop(0, NITER_D, body, 0)
        plsc.subcore_barrier()
        pltpu.sync_copy(
            acc.at[pl.ds(s * DPT, DPT)],
            out_hbm.at[pl.ds(c * N_PAD + s * DPT, DPT)],
        )

    return deg_kernel


def _make_agg_kernel(D):
    mesh = plsc.VectorSubcoreMesh(core_axis_name="c", subcore_axis_name="s")
    NBUF = 3 if D == 128 else 6
    NSUPER = NITER // NBUF

    @functools.partial(
        pl.kernel,
        mesh=mesh,
        compiler_params=_SC_PARAMS,
        out_type=jax.ShapeDtypeStruct((NCORES * N_ACC, D), jnp.float32),
        scratch_types=[
            pltpu.VMEM_SHARED((N_ACC, D), jnp.float32),
            pltpu.VMEM((NITER, CHUNK), jnp.int32),
            pltpu.VMEM((NITER, CHUNK), jnp.int32),
            pltpu.VMEM((NBUF, CHUNK, D), jnp.float32),
            pltpu.SemaphoreType.DMA((NBUF,)),
        ],
    )
    def agg_kernel(xs_hbm, srcr_hbm, dstr_hbm, zeros_hbm, out_hbm,
                   acc, src2d, dst2d, rows_v, gsem):
        c = lax.axis_index("c")
        s = lax.axis_index("s")
        w = c * NSUB + s
        pltpu.sync_copy(zeros_hbm, acc.at[pl.ds(s * SPT, SPT)])
        pltpu.sync_copy(srcr_hbm.at[pl.ds(w * NITER, NITER)], src2d)
        pltpu.sync_copy(dstr_hbm.at[pl.ds(w * NITER, NITER)], dst2d)
        plsc.subcore_barrier()

        def gather(i, b):
            return pltpu.make_async_copy(
                xs_hbm.at[src2d.at[i]], rows_v.at[b], gsem.at[b])

        for b in range(NBUF):
            gather(b, b).start()

        def body(g, carry):
            base = g * NBUF
            for b in range(NBUF):
                i = base + b
                gather(i, b).wait()
                pltpu.sync_copy(rows_v.at[b], acc.at[dst2d.at[i]], add=True)
                gather(i + NBUF, b).start()
            return carry

        lax.fori_loop(0, NSUPER - 1, body, 0)
        base = NITER - NBUF
        for b in range(NBUF):
            gather(base + b, b).wait()
            pltpu.sync_copy(rows_v.at[b], acc.at[dst2d.at[base + b]], add=True)
        plsc.subcore_barrier()
        pltpu.sync_copy(
            acc.at[pl.ds(s * SPT, SPT)],
            out_hbm.at[pl.ds(c * N_ACC + s * SPT, SPT)],
        )

    return agg_kernel


_deg_call = _make_deg_kernel()
_agg128_call = _make_agg_kernel(HIDDEN)
_agg64_call = _make_agg_kernel(CLASSES)

BM = 2000
GRID = N // BM


def _layer1_body(x_ref, w_ref, d0_ref, d1_ref, xs_ref, dinv_ref):
    deg = d0_ref[...] + d1_ref[...] + 1.0
    dinv = lax.rsqrt(deg)
    h = jnp.dot(x_ref[...], w_ref[...], preferred_element_type=jnp.float32)
    xs_ref[...] = h * dinv
    dinv_ref[...] = dinv


def _tc_layer1(x, W1, d0, d1):
    return pl.pallas_call(
        _layer1_body,
        grid=(GRID,),
        in_specs=[
            pl.BlockSpec((BM, D_IN), lambda i: (i, 0)),
            pl.BlockSpec((D_IN, HIDDEN), lambda i: (0, 0)),
            pl.BlockSpec((BM, 1), lambda i: (i, 0)),
            pl.BlockSpec((BM, 1), lambda i: (i, 0)),
        ],
        out_specs=[
            pl.BlockSpec((BM, HIDDEN), lambda i: (i, 0)),
            pl.BlockSpec((BM, 1), lambda i: (i, 0)),
        ],
        out_shape=[
            jax.ShapeDtypeStruct((N, HIDDEN), jnp.float32),
            jax.ShapeDtypeStruct((N, 1), jnp.float32),
        ],
    )(x, W1, d0, d1)


def _layer2_body(a0_ref, a1_ref, xs_ref, dinv_ref, b_ref, w_ref, out_ref):
    dinv = dinv_ref[...]
    z = dinv * (a0_ref[...] + a1_ref[...] + xs_ref[...]) + b_ref[...]
    o = jnp.maximum(z, 0.0)
    h2 = jnp.dot(o, w_ref[...], preferred_element_type=jnp.float32)
    out_ref[...] = h2 * dinv


def _tc_layer2(a0, a1, xs1, dinv, b1, W2):
    return pl.pallas_call(
        _layer2_body,
        grid=(GRID,),
        in_specs=[
            pl.BlockSpec((BM, HIDDEN), lambda i: (i, 0)),
            pl.BlockSpec((BM, HIDDEN), lambda i: (i, 0)),
            pl.BlockSpec((BM, HIDDEN), lambda i: (i, 0)),
            pl.BlockSpec((BM, 1), lambda i: (i, 0)),
            pl.BlockSpec((1, HIDDEN), lambda i: (0, 0)),
            pl.BlockSpec((HIDDEN, CLASSES), lambda i: (0, 0)),
        ],
        out_specs=pl.BlockSpec((BM, CLASSES), lambda i: (i, 0)),
        out_shape=jax.ShapeDtypeStruct((N, CLASSES), jnp.float32),
    )(a0, a1, xs1, dinv, b1, W2)


def _final_body(a0_ref, a1_ref, xs_ref, dinv_ref, b_ref, out_ref):
    z = dinv_ref[...] * (a0_ref[...] + a1_ref[...] + xs_ref[...]) + b_ref[...]
    out_ref[...] = z


def _tc_final(a0, a1, xs2, dinv, b2):
    return pl.pallas_call(
        _final_body,
        grid=(GRID,),
        in_specs=[
            pl.BlockSpec((BM, CLASSES), lambda i: (i, 0)),
            pl.BlockSpec((BM, CLASSES), lambda i: (i, 0)),
            pl.BlockSpec((BM, CLASSES), lambda i: (i, 0)),
            pl.BlockSpec((BM, 1), lambda i: (i, 0)),
            pl.BlockSpec((1, CLASSES), lambda i: (0, 0)),
        ],
        out_specs=pl.BlockSpec((BM, CLASSES), lambda i: (i, 0)),
        out_shape=jax.ShapeDtypeStruct((N, CLASSES), jnp.float32),
    )(a0, a1, xs2, dinv, b2)


def kernel(data, edge_index, W1, b1, W2, b2):
    ei = edge_index.astype(jnp.int32)
    pad = jnp.full((E_PAD - E,), N, jnp.int32)
    src = jnp.concatenate([ei[0], pad]).reshape(E_PAD // CHUNK, CHUNK)
    dst_flat = jnp.concatenate([ei[1], pad])
    dst = dst_flat.reshape(E_PAD // CHUNK, CHUNK)
    dst_deg = dst_flat.reshape(E_PAD // CHUNK_D, CHUNK_D)
    zrows128 = jnp.zeros((N_ACC - N, HIDDEN), jnp.float32)
    zrows64 = jnp.zeros((N_ACC - N, CLASSES), jnp.float32)

    zeros_deg = jnp.zeros((DPT,), jnp.float32)
    zeros128 = jnp.zeros((SPT, HIDDEN), jnp.float32)
    zeros64 = jnp.zeros((SPT, CLASSES), jnp.float32)

    deg_parts = _deg_call(dst_deg, zeros_deg)
    d0 = deg_parts[0 * N_PAD:0 * N_PAD + N].reshape(N, 1)
    d1 = deg_parts[1 * N_PAD:1 * N_PAD + N].reshape(N, 1)

    xs1, dinv = _tc_layer1(data, W1, d0, d1)

    agg1 = _agg128_call(jnp.concatenate([xs1, zrows128]), src, dst,
                        zeros128)
    xs2 = _tc_layer2(agg1[:N], agg1[N_ACC:N_ACC + N], xs1, dinv,
                     b1.reshape(1, HIDDEN), W2)

    agg2 = _agg64_call(jnp.concatenate([xs2, zrows64]), src, dst,
                       zeros64)
    out = _tc_final(agg2[:N], agg2[N_ACC:N_ACC + N], xs2, dinv,
                    b2.reshape(1, CLASSES))
    return out


# speedup vs baseline: 1.6978x; 1.6060x over previous
import functools
import jax
import jax.numpy as jnp
from jax import lax
from jax.experimental import pallas as pl
from jax.experimental.pallas import tpu as pltpu
from jax.experimental.pallas import tpu_sc as plsc

N = 10000
E = 320000
D_IN = 128
HIDDEN = 128
CLASSES = 64

NCORES = 2
NSUB = 16
NW = NCORES * NSUB
CHUNK = 80
E_PER = E // NW
NITER = E_PER // CHUNK
CHUNK_D = 80
NITER_D = NITER
SPT = N // NSUB
N_ACC = N
N_PAD = 10240
DPT = N_PAD // NSUB


_SC_PARAMS = pltpu.CompilerParams(use_tc_tiling_on_sc=False)


def _make_deg_kernel():
    mesh = plsc.VectorSubcoreMesh(core_axis_name="c", subcore_axis_name="s")

    @functools.partial(
        pl.kernel,
        mesh=mesh,
        compiler_params=_SC_PARAMS,
        out_type=jax.ShapeDtypeStruct((NCORES * N_PAD,), jnp.float32),
        scratch_types=[
            pltpu.VMEM_SHARED((N_PAD,), jnp.float32),
            pltpu.VMEM((NITER_D, CHUNK_D), jnp.int32),
            pltpu.VMEM((CHUNK_D,), jnp.float32),
        ],
    )
    def deg_kernel(dstr_hbm, zeros_hbm, out_hbm, acc, dst2d, ones_v):
        c = lax.axis_index("c")
        s = lax.axis_index("s")
        w = c * NSUB + s
        pltpu.sync_copy(zeros_hbm, acc.at[pl.ds(s * DPT, DPT)])
        pltpu.sync_copy(dstr_hbm.at[pl.ds(w * NITER_D, NITER_D)], dst2d)
        for j in range(CHUNK_D // 16):
            ones_v[pl.ds(j * 16, 16)] = jnp.full((16,), 1.0, jnp.float32)
        plsc.subcore_barrier()

        def body(i, carry):
            pltpu.sync_copy(ones_v, acc.at[dst2d.at[i]], add=True)
            return carry

        lax.fori_loop(0, NITER_D, body, 0)
        plsc.subcore_barrier()
        pltpu.sync_copy(
            acc.at[pl.ds(s * DPT, DPT)],
            out_hbm.at[pl.ds(c * N_PAD + s * DPT, DPT)],
        )

    return deg_kernel


def _make_agg_kernel(D):
    mesh = plsc.VectorSubcoreMesh(core_axis_name="c", subcore_axis_name="s")
    NBUF = 3 if D == 128 else 6
    GMAIN = (NITER - NBUF) // NBUF

    @functools.partial(
        pl.kernel,
        mesh=mesh,
        compiler_params=_SC_PARAMS,
        out_type=jax.ShapeDtypeStruct((NCORES * N_ACC, D), jnp.float32),
        scratch_types=[
            pltpu.VMEM_SHARED((N_ACC, D), jnp.float32),
            pltpu.VMEM((NITER, CHUNK), jnp.int32),
            pltpu.VMEM((NITER, CHUNK), jnp.int32),
            pltpu.VMEM((NBUF, CHUNK, D), jnp.float32),
            pltpu.SemaphoreType.DMA((NBUF,)),
        ],
    )
    def agg_kernel(xs_hbm, srcr_hbm, dstr_hbm, zeros_hbm, out_hbm,
                   acc, src2d, dst2d, rows_v, gsem):
        c = lax.axis_index("c")
        s = lax.axis_index("s")
        w = c * NSUB + s
        pltpu.sync_copy(zeros_hbm, acc.at[pl.ds(s * SPT, SPT)])
        pltpu.sync_copy(srcr_hbm.at[pl.ds(w * NITER, NITER)], src2d)
        pltpu.sync_copy(dstr_hbm.at[pl.ds(w * NITER, NITER)], dst2d)
        plsc.subcore_barrier()

        def gather(i, b):
            return pltpu.make_async_copy(
                xs_hbm.at[src2d.at[i]], rows_v.at[b], gsem.at[b])

        for b in range(NBUF):
            gather(b, b).start()

        def body(g, carry):
            base = g * NBUF
            for b in range(NBUF):
                i = base + b
                gather(i, b).wait()
                pltpu.sync_copy(rows_v.at[b], acc.at[dst2d.at[i]], add=True)
                gather(i + NBUF, b).start()
            return carry

        lax.fori_loop(0, GMAIN, body, 0)
        for i in range(GMAIN * NBUF, NITER):
            b = i % NBUF
            gather(i, b).wait()
            pltpu.sync_copy(rows_v.at[b], acc.at[dst2d.at[i]], add=True)
            if i + NBUF < NITER:
                gather(i + NBUF, b).start()
        plsc.subcore_barrier()
        pltpu.sync_copy(
            acc.at[pl.ds(s * SPT, SPT)],
            out_hbm.at[pl.ds(c * N_ACC + s * SPT, SPT)],
        )

    return agg_kernel


_deg_call = _make_deg_kernel()
_agg128_call = _make_agg_kernel(HIDDEN)
_agg64_call = _make_agg_kernel(CLASSES)

BM = 2000
GRID = N // BM


def _layer1_body(x_ref, w_ref, d0_ref, d1_ref, xs_ref, dinv_ref):
    deg = d0_ref[...] + d1_ref[...] + 1.0
    dinv = lax.rsqrt(deg)
    h = jnp.dot(x_ref[...], w_ref[...], preferred_element_type=jnp.float32)
    xs_ref[...] = h * dinv
    dinv_ref[...] = dinv


def _tc_layer1(x, W1, d0, d1):
    return pl.pallas_call(
        _layer1_body,
        grid=(GRID,),
        in_specs=[
            pl.BlockSpec((BM, D_IN), lambda i: (i, 0)),
            pl.BlockSpec((D_IN, HIDDEN), lambda i: (0, 0)),
            pl.BlockSpec((BM, 1), lambda i: (i, 0)),
            pl.BlockSpec((BM, 1), lambda i: (i, 0)),
        ],
        out_specs=[
            pl.BlockSpec((BM, HIDDEN), lambda i: (i, 0)),
            pl.BlockSpec((BM, 1), lambda i: (i, 0)),
        ],
        out_shape=[
            jax.ShapeDtypeStruct((N, HIDDEN), jnp.float32),
            jax.ShapeDtypeStruct((N, 1), jnp.float32),
        ],
    )(x, W1, d0, d1)


def _layer2_body(a0_ref, a1_ref, xs_ref, dinv_ref, b_ref, w_ref, out_ref):
    dinv = dinv_ref[...]
    z = dinv * (a0_ref[...] + a1_ref[...] + xs_ref[...]) + b_ref[...]
    o = jnp.maximum(z, 0.0)
    h2 = jnp.dot(o, w_ref[...], preferred_element_type=jnp.float32)
    out_ref[...] = h2 * dinv


def _tc_layer2(a0, a1, xs1, dinv, b1, W2):
    return pl.pallas_call(
        _layer2_body,
        grid=(GRID,),
        in_specs=[
            pl.BlockSpec((BM, HIDDEN), lambda i: (i, 0)),
            pl.BlockSpec((BM, HIDDEN), lambda i: (i, 0)),
            pl.BlockSpec((BM, HIDDEN), lambda i: (i, 0)),
            pl.BlockSpec((BM, 1), lambda i: (i, 0)),
            pl.BlockSpec((1, HIDDEN), lambda i: (0, 0)),
            pl.BlockSpec((HIDDEN, CLASSES), lambda i: (0, 0)),
        ],
        out_specs=pl.BlockSpec((BM, CLASSES), lambda i: (i, 0)),
        out_shape=jax.ShapeDtypeStruct((N, CLASSES), jnp.float32),
    )(a0, a1, xs1, dinv, b1, W2)


def _final_body(a0_ref, a1_ref, xs_ref, dinv_ref, b_ref, out_ref):
    z = dinv_ref[...] * (a0_ref[...] + a1_ref[...] + xs_ref[...]) + b_ref[...]
    out_ref[...] = z


def _tc_final(a0, a1, xs2, dinv, b2):
    return pl.pallas_call(
        _final_body,
        grid=(GRID,),
        in_specs=[
            pl.BlockSpec((BM, CLASSES), lambda i: (i, 0)),
            pl.BlockSpec((BM, CLASSES), lambda i: (i, 0)),
            pl.BlockSpec((BM, CLASSES), lambda i: (i, 0)),
            pl.BlockSpec((BM, 1), lambda i: (i, 0)),
            pl.BlockSpec((1, CLASSES), lambda i: (0, 0)),
        ],
        out_specs=pl.BlockSpec((BM, CLASSES), lambda i: (i, 0)),
        out_shape=jax.ShapeDtypeStruct((N, CLASSES), jnp.float32),
    )(a0, a1, xs2, dinv, b2)


def kernel(data, edge_index, W1, b1, W2, b2):
    ei = edge_index.astype(jnp.int32)
    src = ei[0].reshape(E // CHUNK, CHUNK)
    dst = ei[1].reshape(E // CHUNK, CHUNK)

    zeros_deg = jnp.zeros((DPT,), jnp.float32)
    zeros128 = jnp.zeros((SPT, HIDDEN), jnp.float32)
    zeros64 = jnp.zeros((SPT, CLASSES), jnp.float32)

    deg_parts = _deg_call(dst, zeros_deg)
    d0 = deg_parts[0 * N_PAD:0 * N_PAD + N].reshape(N, 1)
    d1 = deg_parts[1 * N_PAD:1 * N_PAD + N].reshape(N, 1)

    xs1, dinv = _tc_layer1(data, W1, d0, d1)

    agg1 = _agg128_call(xs1, src, dst, zeros128)
    xs2 = _tc_layer2(agg1[:N], agg1[N_ACC:N_ACC + N], xs1, dinv,
                     b1.reshape(1, HIDDEN), W2)

    agg2 = _agg64_call(xs2, src, dst, zeros64)
    out = _tc_final(agg2[:N], agg2[N_ACC:N_ACC + N], xs2, dinv,
                    b2.reshape(1, CLASSES))
    return out
